# baseline (speedup 1.0000x reference)
import jax
import jax.numpy as jnp
from jax.experimental import pallas as pl
from jax.experimental.pallas import tpu as pltpu

_T, _B, _DIM, _K = 2048, 8, 256, 1024
_R = 2048
_N = _T * _B
_G = _N // _R


def _vq_kernel(x_ref, m_ref, emb_ref, embt_ref,
               q_ref, ind_ref, diff_ref, eu_ref,
               cnt_acc):
    i = pl.program_id(0)

    @pl.when(i == 0)
    def _init():
        cnt_acc[...] = jnp.zeros_like(cnt_acc)

    x = x_ref[...]
    emb = emb_ref[...]
    m = m_ref[0, 0, :]

    s = jax.lax.dot_general(x, emb, (((1,), (0,)), ((), ())),
                            preferred_element_type=jnp.float32)
    x2 = jnp.sum(x * x, axis=1, keepdims=True)
    e2 = jnp.sum(emb * emb, axis=0, keepdims=True)
    dist = (x2 - 2.0 * s) + e2

    iota_f = jax.lax.broadcasted_iota(jnp.int32, (_R, _K), 1).astype(jnp.float32)
    dmin = jnp.min(dist, axis=1, keepdims=True)
    cand = jnp.where(dist <= dmin, iota_f, float(_K))
    ind_f = jnp.min(cand, axis=1, keepdims=True)

    onehot_b = (iota_f == ind_f).astype(jnp.bfloat16)
    q = jax.lax.dot_general(onehot_b, embt_ref[...],
                            (((1,), (0,)), ((), ())),
                            preferred_element_type=jnp.float32)
    q_ref[...] = q * m[:, None]
    ind_ref[...] = ind_f.astype(jnp.int32)

    stats = jnp.concatenate(
        [m[None, :], (m * dmin[:, 0])[None, :]], axis=0).astype(jnp.bfloat16)
    cnt_acc[...] += jax.lax.dot_general(
        stats, onehot_b,
        (((1,), (0,)), ((), ())), preferred_element_type=jnp.float32)

    @pl.when(i == _G - 1)
    def _fin():
        cnt = cnt_acc[0, :]
        msum = jnp.sum(cnt)
        diff_ref[...] = jnp.broadcast_to(
            jnp.sum(cnt_acc[1, :]) / float(_N * _DIM), (1, 1))
        mm = cnt / jnp.maximum(msum, 1.0)
        eu_ref[...] = jnp.broadcast_to(1.0 / jnp.sum(mm * mm), (1, 1))


def kernel(input, input_mask, embed):
    x = input.reshape(_N, _DIM)
    maskf = input_mask.reshape(_G, 1, _R).astype(jnp.float32)

    q, ind3, diff, eu = pl.pallas_call(
        _vq_kernel,
        grid=(_G,),
        in_specs=[
            pl.BlockSpec((_R, _DIM), lambda i: (i, 0)),
            pl.BlockSpec((1, 1, _R), lambda i: (i, 0, 0)),
            pl.BlockSpec((_DIM, _K), lambda i: (0, 0)),
            pl.BlockSpec((_K, _DIM), lambda i: (0, 0)),
        ],
        out_specs=[
            pl.BlockSpec((_R, _DIM), lambda i: (i, 0)),
            pl.BlockSpec((_R, 1), lambda i: (i, 0)),
            pl.BlockSpec((1, 1), lambda i: (0, 0)),
            pl.BlockSpec((1, 1), lambda i: (0, 0)),
        ],
        out_shape=[
            jax.ShapeDtypeStruct((_N, _DIM), jnp.float32),
            jax.ShapeDtypeStruct((_N, 1), jnp.int32),
            jax.ShapeDtypeStruct((1, 1), jnp.float32),
            jax.ShapeDtypeStruct((1, 1), jnp.float32),
        ],
        scratch_shapes=[
            pltpu.VMEM((2, _K), jnp.float32),
        ],
    )(x, maskf, embed, embed.T.astype(jnp.bfloat16))

    quantize = q.reshape(_T, _B, _DIM)
    embed_ind = ind3.reshape(_N)
    return quantize, diff[0, 0], embed_ind, eu[0, 0]


# speedup vs baseline: 1.0347x; 1.0347x over previous
import jax
import jax.numpy as jnp
from jax.experimental import pallas as pl
from jax.experimental.pallas import tpu as pltpu

_T, _B, _DIM, _K = 2048, 8, 256, 1024
_R = 2048
_N = _T * _B
_G = _N // _R


def _vq_kernel(x_ref, m_ref, emb_ref, embt_ref,
               q_ref, ind_ref, diff_ref, eu_ref,
               cnt_acc):
    i = pl.program_id(0)

    @pl.when(i == 0)
    def _init():
        cnt_acc[...] = jnp.zeros_like(cnt_acc)

    x = x_ref[...]
    emb = emb_ref[...]
    m = m_ref[0, 0, :]

    s = jax.lax.dot_general(x, emb, (((1,), (0,)), ((), ())),
                            preferred_element_type=jnp.float32)
    x2 = jnp.sum(x * x, axis=1, keepdims=True)
    e2 = jnp.sum(emb * emb, axis=0, keepdims=True)
    dist = (x2 - 2.0 * s) + e2

    iota_f = jax.lax.broadcasted_iota(jnp.int32, (_R, _K), 1).astype(jnp.float32)
    dmin = jnp.min(dist, axis=1, keepdims=True)
    cand = jnp.where(dist <= dmin, iota_f, float(_K))
    ind_f = jnp.min(cand, axis=1, keepdims=True)
    ind = ind_f[:, 0].astype(jnp.int32)

    onehot_b = (iota_f == ind_f).astype(jnp.bfloat16)
    q = jax.lax.dot_general(onehot_b, embt_ref[...],
                            (((1,), (0,)), ((), ())),
                            preferred_element_type=jnp.float32)
    q_ref[...] = q * m[:, None]
    ind_ref[0, 0, :] = ind

    stats = jnp.concatenate(
        [m[None, :], (m * dmin[:, 0])[None, :]], axis=0).astype(jnp.bfloat16)
    cnt_acc[...] += jax.lax.dot_general(
        stats, onehot_b,
        (((1,), (0,)), ((), ())), preferred_element_type=jnp.float32)

    @pl.when(i == _G - 1)
    def _fin():
        cnt = cnt_acc[0, :]
        msum = jnp.sum(cnt)
        diff_ref[...] = jnp.broadcast_to(
            jnp.sum(cnt_acc[1, :]) / float(_N * _DIM), (1, 1))
        mm = cnt / jnp.maximum(msum, 1.0)
        eu_ref[...] = jnp.broadcast_to(1.0 / jnp.sum(mm * mm), (1, 1))


def kernel(input, input_mask, embed):
    x = input.reshape(_N, _DIM)
    maskf = input_mask.reshape(_G, 1, _R).astype(jnp.float32)

    q, ind3, diff, eu = pl.pallas_call(
        _vq_kernel,
        grid=(_G,),
        in_specs=[
            pl.BlockSpec((_R, _DIM), lambda i: (i, 0)),
            pl.BlockSpec((1, 1, _R), lambda i: (i, 0, 0)),
            pl.BlockSpec((_DIM, _K), lambda i: (0, 0)),
            pl.BlockSpec((_K, _DIM), lambda i: (0, 0)),
        ],
        out_specs=[
            pl.BlockSpec((_R, _DIM), lambda i: (i, 0)),
            pl.BlockSpec((1, 1, _R), lambda i: (i, 0, 0)),
            pl.BlockSpec((1, 1), lambda i: (0, 0)),
            pl.BlockSpec((1, 1), lambda i: (0, 0)),
        ],
        out_shape=[
            jax.ShapeDtypeStruct((_N, _DIM), jnp.float32),
            jax.ShapeDtypeStruct((_G, 1, _R), jnp.int32),
            jax.ShapeDtypeStruct((1, 1), jnp.float32),
            jax.ShapeDtypeStruct((1, 1), jnp.float32),
        ],
        scratch_shapes=[
            pltpu.VMEM((2, _K), jnp.float32),
        ],
    )(x, maskf, embed, embed.T.astype(jnp.bfloat16))

    quantize = q.reshape(_T, _B, _DIM)
    embed_ind = ind3.reshape(_N)
    return quantize, diff[0, 0], embed_ind, eu[0, 0]


# speedup vs baseline: 1.1260x; 1.0882x over previous
import jax
import jax.numpy as jnp
from jax.experimental import pallas as pl
from jax.experimental.pallas import tpu as pltpu

_T, _B, _DIM, _K = 2048, 8, 256, 1024
_R = 2048
_N = _T * _B
_G = _N // _R


def _vq_kernel(x_ref, m_ref, emb_ref, embt_ref,
               q_ref, ind_ref, diff_ref, eu_ref,
               cnt_acc, dsum_acc, msum_acc):
    i = pl.program_id(0)

    @pl.when(i == 0)
    def _init():
        cnt_acc[...] = jnp.zeros_like(cnt_acc)
        dsum_acc[0] = 0.0
        msum_acc[0] = 0.0

    x = x_ref[...]
    emb = emb_ref[...]
    m = m_ref[0, 0, :]

    s = jax.lax.dot_general(x, emb, (((1,), (0,)), ((), ())),
                            preferred_element_type=jnp.float32)
    x2 = jnp.sum(x * x, axis=1, keepdims=True)
    e2 = jnp.sum(emb * emb, axis=0, keepdims=True)
    dist = (x2 - 2.0 * s) + e2

    iota_f = jax.lax.broadcasted_iota(jnp.int32, (_R, _K), 1).astype(jnp.float32)
    dmin = jnp.min(dist, axis=1, keepdims=True)
    cand = jnp.where(dist <= dmin, iota_f, float(_K))
    ind_f = jnp.min(cand, axis=1)
    ind = ind_f.astype(jnp.int32)

    onehot_b = (iota_f == ind_f[:, None]).astype(jnp.bfloat16)
    q = jax.lax.dot_general(onehot_b, embt_ref[...],
                            (((1,), (0,)), ((), ())),
                            preferred_element_type=jnp.float32)
    q_ref[...] = q * m[:, None]
    ind_ref[0, 0, :] = ind

    cnt_acc[...] += jax.lax.dot_general(
        m.astype(jnp.bfloat16)[None, :], onehot_b,
        (((1,), (0,)), ((), ())), preferred_element_type=jnp.float32)
    dsum_acc[0] += jnp.sum(dmin[:, 0] * m)
    msum_acc[0] += jnp.sum(m)

    @pl.when(i == _G - 1)
    def _fin():
        diff_ref[...] = jnp.broadcast_to(dsum_acc[0] / float(_N * _DIM), (1, 1))
        mm = cnt_acc[...] / jnp.maximum(msum_acc[0], 1.0)
        eu_ref[...] = jnp.broadcast_to(1.0 / jnp.sum(mm * mm), (1, 1))


def kernel(input, input_mask, embed):
    x = input.reshape(_N, _DIM)
    maskf = input_mask.reshape(_G, 1, _R).astype(jnp.float32)

    q, ind3, diff, eu = pl.pallas_call(
        _vq_kernel,
        grid=(_G,),
        in_specs=[
            pl.BlockSpec((_R, _DIM), lambda i: (i, 0)),
            pl.BlockSpec((1, 1, _R), lambda i: (i, 0, 0)),
            pl.BlockSpec((_DIM, _K), lambda i: (0, 0)),
            pl.BlockSpec((_K, _DIM), lambda i: (0, 0)),
        ],
        out_specs=[
            pl.BlockSpec((_R, _DIM), lambda i: (i, 0)),
            pl.BlockSpec((1, 1, _R), lambda i: (i, 0, 0)),
            pl.BlockSpec((1, 1), lambda i: (0, 0)),
            pl.BlockSpec((1, 1), lambda i: (0, 0)),
        ],
        out_shape=[
            jax.ShapeDtypeStruct((_N, _DIM), jnp.float32),
            jax.ShapeDtypeStruct((_G, 1, _R), jnp.int32),
            jax.ShapeDtypeStruct((1, 1), jnp.float32),
            jax.ShapeDtypeStruct((1, 1), jnp.float32),
        ],
        scratch_shapes=[
            pltpu.VMEM((1, _K), jnp.float32),
            pltpu.SMEM((1,), jnp.float32),
            pltpu.SMEM((1,), jnp.float32),
        ],
    )(x, maskf, embed, embed.T.astype(jnp.bfloat16))

    quantize = q.reshape(_T, _B, _DIM)
    embed_ind = ind3.reshape(_N)
    return quantize, diff[0, 0], embed_ind, eu[0, 0]


# speedup vs baseline: 1.2236x; 1.0867x over previous
import jax
import jax.numpy as jnp
from jax.experimental import pallas as pl
from jax.experimental.pallas import tpu as pltpu

_T, _B, _DIM, _K = 2048, 8, 256, 1024
_R = 2048
_N = _T * _B
_G = _N // _R


def _vq_kernel(x_ref, m_ref, emb_ref, embb_ref,
               q_ref, ind_ref, diff_ref, eu_ref,
               cnt_acc, dsum_acc, msum_acc):
    i = pl.program_id(0)

    @pl.when(i == 0)
    def _init():
        cnt_acc[...] = jnp.zeros_like(cnt_acc)
        dsum_acc[0] = 0.0
        msum_acc[0] = 0.0

    x = x_ref[...]
    emb = emb_ref[...]
    m = m_ref[0, 0, :]

    s = jax.lax.dot_general(x, emb, (((1,), (0,)), ((), ())),
                            preferred_element_type=jnp.float32)
    x2 = jnp.sum(x * x, axis=1, keepdims=True)
    e2 = jnp.sum(emb * emb, axis=0, keepdims=True)
    dist = (x2 - 2.0 * s) + e2

    iota_f = jax.lax.broadcasted_iota(jnp.int32, (_R, _K), 1).astype(jnp.float32)
    dmin = jnp.min(dist, axis=1, keepdims=True)
    cand = jnp.where(dist <= dmin, iota_f, float(_K))
    ind_f = jnp.min(cand, axis=1, keepdims=True)
    ind_row = jnp.transpose(ind_f)

    onehot_b = (iota_f == ind_f).astype(jnp.bfloat16)
    q = jax.lax.dot_general(onehot_b, embb_ref[...],
                            (((1,), (1,)), ((), ())),
                            preferred_element_type=jnp.float32)
    q_ref[...] = q * m[:, None]
    ind_ref[0, 0, :] = ind_row[0, :].astype(jnp.int32)

    cnt_acc[...] += jax.lax.dot_general(
        m.astype(jnp.bfloat16)[None, :], onehot_b,
        (((1,), (0,)), ((), ())), preferred_element_type=jnp.float32)
    dsum_acc[0] += jnp.sum(dmin[:, 0] * m)
    msum_acc[0] += jnp.sum(m)

    @pl.when(i == _G - 1)
    def _fin():
        diff_ref[...] = jnp.broadcast_to(dsum_acc[0] / float(_N * _DIM), (1, 1))
        mm = cnt_acc[...] / jnp.maximum(msum_acc[0], 1.0)
        eu_ref[...] = jnp.broadcast_to(1.0 / jnp.sum(mm * mm), (1, 1))


def kernel(input, input_mask, embed):
    x = input.reshape(_N, _DIM)
    maskf = input_mask.reshape(_G, 1, _R).astype(jnp.float32)

    q, ind3, diff, eu = pl.pallas_call(
        _vq_kernel,
        grid=(_G,),
        in_specs=[
            pl.BlockSpec((_R, _DIM), lambda i: (i, 0)),
            pl.BlockSpec((1, 1, _R), lambda i: (i, 0, 0)),
            pl.BlockSpec((_DIM, _K), lambda i: (0, 0)),
            pl.BlockSpec((_DIM, _K), lambda i: (0, 0)),
        ],
        out_specs=[
            pl.BlockSpec((_R, _DIM), lambda i: (i, 0)),
            pl.BlockSpec((1, 1, _R), lambda i: (i, 0, 0)),
            pl.BlockSpec((1, 1), lambda i: (0, 0)),
            pl.BlockSpec((1, 1), lambda i: (0, 0)),
        ],
        out_shape=[
            jax.ShapeDtypeStruct((_N, _DIM), jnp.float32),
            jax.ShapeDtypeStruct((_G, 1, _R), jnp.int32),
            jax.ShapeDtypeStruct((1, 1), jnp.float32),
            jax.ShapeDtypeStruct((1, 1), jnp.float32),
        ],
        scratch_shapes=[
            pltpu.VMEM((1, _K), jnp.float32),
            pltpu.SMEM((1,), jnp.float32),
            pltpu.SMEM((1,), jnp.float32),
        ],
    )(x, maskf, embed, embed.astype(jnp.bfloat16))

    quantize = q.reshape(_T, _B, _DIM)
    embed_ind = ind3.reshape(_N)
    return quantize, diff[0, 0], embed_ind, eu[0, 0]


# speedup vs baseline: 1.2289x; 1.0043x over previous
import jax
import jax.numpy as jnp
from jax.experimental import pallas as pl
from jax.experimental.pallas import tpu as pltpu

_T, _B, _DIM, _K = 2048, 8, 256, 1024
_R = 2048
_N = _T * _B
_G = _N // _R


def _vq_kernel(x_ref, m_ref, emb_ref, embb_ref,
               q_ref, ind_ref, diff_ref, eu_ref,
               cnt_acc, dvec_acc):
    i = pl.program_id(0)

    @pl.when(i == 0)
    def _init():
        cnt_acc[...] = jnp.zeros_like(cnt_acc)
        dvec_acc[...] = jnp.zeros_like(dvec_acc)

    x = x_ref[...]
    emb = emb_ref[...]
    m = m_ref[0, 0, :]

    s = jax.lax.dot_general(x, emb, (((1,), (0,)), ((), ())),
                            preferred_element_type=jnp.float32)
    x2 = jnp.sum(x * x, axis=1, keepdims=True)
    e2 = jnp.sum(emb * emb, axis=0, keepdims=True)
    dist = (x2 - 2.0 * s) + e2

    iota_f = jax.lax.broadcasted_iota(jnp.int32, (_R, _K), 1).astype(jnp.float32)
    dmin = jnp.min(dist, axis=1, keepdims=True)
    cand = jnp.where(dist <= dmin, iota_f, float(_K))
    ind_f = jnp.min(cand, axis=1, keepdims=True)
    ind_row = jnp.transpose(ind_f)

    onehot_b = (iota_f == ind_f).astype(jnp.bfloat16)
    q = jax.lax.dot_general(onehot_b, embb_ref[...],
                            (((1,), (1,)), ((), ())),
                            preferred_element_type=jnp.float32)
    q_ref[...] = q * m[:, None]
    ind_ref[0, 0, :] = ind_row[0, :].astype(jnp.int32)

    cnt_acc[...] += jax.lax.dot_general(
        m.astype(jnp.bfloat16)[None, :], onehot_b,
        (((1,), (0,)), ((), ())), preferred_element_type=jnp.float32)
    dvec_acc[...] += jnp.transpose(dmin) * m[None, :]

    @pl.when(i == _G - 1)
    def _fin():
        diff_ref[...] = jnp.broadcast_to(
            jnp.sum(dvec_acc[...]) / float(_N * _DIM), (1, 1))
        mm = cnt_acc[...] / jnp.maximum(jnp.sum(cnt_acc[...]), 1.0)
        eu_ref[...] = jnp.broadcast_to(1.0 / jnp.sum(mm * mm), (1, 1))


def kernel(input, input_mask, embed):
    x = input.reshape(_N, _DIM)
    maskf = input_mask.reshape(_G, 1, _R).astype(jnp.float32)

    q, ind3, diff, eu = pl.pallas_call(
        _vq_kernel,
        grid=(_G,),
        in_specs=[
            pl.BlockSpec((_R, _DIM), lambda i: (i, 0)),
            pl.BlockSpec((1, 1, _R), lambda i: (i, 0, 0)),
            pl.BlockSpec((_DIM, _K), lambda i: (0, 0)),
            pl.BlockSpec((_DIM, _K), lambda i: (0, 0)),
        ],
        out_specs=[
            pl.BlockSpec((_R, _DIM), lambda i: (i, 0)),
            pl.BlockSpec((1, 1, _R), lambda i: (i, 0, 0)),
            pl.BlockSpec((1, 1), lambda i: (0, 0)),
            pl.BlockSpec((1, 1), lambda i: (0, 0)),
        ],
        out_shape=[
            jax.ShapeDtypeStruct((_N, _DIM), jnp.float32),
            jax.ShapeDtypeStruct((_G, 1, _R), jnp.int32),
            jax.ShapeDtypeStruct((1, 1), jnp.float32),
            jax.ShapeDtypeStruct((1, 1), jnp.float32),
        ],
        scratch_shapes=[
            pltpu.VMEM((1, _K), jnp.float32),
            pltpu.VMEM((1, _R), jnp.float32),
        ],
    )(x, maskf, embed, embed.astype(jnp.bfloat16))

    quantize = q.reshape(_T, _B, _DIM)
    embed_ind = ind3.reshape(_N)
    return quantize, diff[0, 0], embed_ind, eu[0, 0]
